# baseline (speedup 1.0000x reference)
import functools

import jax
import jax.numpy as jnp
from jax import lax
from jax.experimental import pallas as pl
from jax.experimental.pallas import tpu as pltpu
from jax.experimental.pallas import tpu_sc as plsc

N_NODES = 10000
D = 128
E_TOTAL = 320000
NC = 2
NS = 16
NW = NC * NS
EPW = E_TOTAL // NW
CHUNK = 80
NSTEP = EPW // CHUNK
ROW_STRIDE = 624
COPY_ROWS = 640

_mesh = plsc.VectorSubcoreMesh(core_axis_name="c", subcore_axis_name="s")


def _make_pass_kernel(width):
    @functools.partial(
        pl.kernel,
        out_type=jax.ShapeDtypeStruct((NC * N_NODES, width), jnp.float32),
        mesh=_mesh,
        scratch_types=[
            pltpu.VMEM((EPW,), jnp.int32),
            pltpu.VMEM((NSTEP, CHUNK), jnp.int32),
            pltpu.VMEM((CHUNK, width), jnp.float32),
            pltpu.VMEM((CHUNK, width), jnp.float32),
            pltpu.VMEM_SHARED((N_NODES, width), jnp.float32),
            pltpu.SemaphoreType.DMA,
            pltpu.SemaphoreType.DMA,
        ],
    )
    def pass_kernel(table_hbm, src_hbm, dst_hbm, zrows_hbm, out_hbm,
                    sidx, didx, rows0, rows1, accum, sem0, sem1):
        c = lax.axis_index("c")
        s = lax.axis_index("s")
        wid = s * NC + c
        r0 = pl.multiple_of(s * ROW_STRIDE, 8)

        pltpu.sync_copy(zrows_hbm.at[pl.ds(r0, COPY_ROWS)],
                        accum.at[pl.ds(r0, COPY_ROWS)])
        base = pl.multiple_of(wid * EPW, 8)
        pltpu.sync_copy(src_hbm.at[pl.ds(base, EPW)], sidx)
        pltpu.sync_copy(dst_hbm.at[wid], didx)
        plsc.subcore_barrier()

        def start_gather(j, rows, sem):
            pltpu.async_copy(
                table_hbm.at[sidx.at[pl.ds(j * CHUNK, CHUNK)]], rows, sem)

        def wait_gather(rows, sem):
            pltpu.make_async_copy(table_hbm.at[pl.ds(0, CHUNK)], rows,
                                  sem).wait()

        start_gather(0, rows0, sem0)

        def pair(k, carry):
            j0 = k * 2
            wait_gather(rows0, sem0)
            start_gather(j0 + 1, rows1, sem1)
            pltpu.sync_copy(rows0, accum.at[didx.at[j0]], add=True)
            wait_gather(rows1, sem1)
            start_gather(j0 + 2, rows0, sem0)
            pltpu.sync_copy(rows1, accum.at[didx.at[j0 + 1]], add=True)
            return carry

        lax.fori_loop(0, NSTEP // 2, pair, 0)
        wait_gather(rows0, sem0)
        pltpu.sync_copy(rows0, accum.at[didx.at[NSTEP - 1]], add=True)

        plsc.subcore_barrier()
        pltpu.sync_copy(
            accum.at[pl.ds(r0, COPY_ROWS)],
            out_hbm.at[pl.ds(c * N_NODES + r0, COPY_ROWS)])

    return pass_kernel


_message_kernel = _make_pass_kernel(D)

DEG_HI = 80
CH_E = 16000
NCH = E_TOTAL // CH_E


def _bincount_body(idx_ref, deg_ref):
    i = pl.program_id(0)

    @pl.when(i == 0)
    def _():
        deg_ref[...] = jnp.zeros_like(deg_ref)

    idx_col = idx_ref[...]
    hi = idx_col // 128
    lo = idx_col - hi * 128
    hi16 = hi.astype(jnp.int16)
    lo16 = lo.astype(jnp.int16)
    a = (lo16 == lax.broadcasted_iota(jnp.int16, (CH_E, 128), 1))
    bt = (hi16 == lax.broadcasted_iota(jnp.int16, (CH_E, DEG_HI), 1))
    deg_ref[...] += lax.dot_general(
        bt.astype(jnp.bfloat16), a.astype(jnp.bfloat16),
        (((0,), (0,)), ((), ())), preferred_element_type=jnp.float32)


_bincount_tc = pl.pallas_call(
    _bincount_body,
    grid=(NCH,),
    in_specs=[pl.BlockSpec((CH_E, 1), lambda i: (i, 0))],
    out_specs=pl.BlockSpec((DEG_HI, 128), lambda i: (0, 0)),
    out_shape=jax.ShapeDtypeStruct((DEG_HI, 128), jnp.float32),
)


def _scale_body(node_ref, deg_ref, out_ref):
    deg = jnp.maximum(deg_ref[...], 1.0)
    out_ref[...] = node_ref[...] * lax.rsqrt(deg)


_scale = pl.pallas_call(
    _scale_body,
    out_shape=jax.ShapeDtypeStruct((N_NODES, D), jnp.float32),
)


def _finish_body(part_ref, deg_ref, out_ref):
    total = part_ref[0] + part_ref[1]
    deg = jnp.maximum(deg_ref[...], 1.0)
    out_ref[...] = total * lax.rsqrt(deg)


_finish = pl.pallas_call(
    _finish_body,
    out_shape=jax.ShapeDtypeStruct((N_NODES, D), jnp.float32),
)


def kernel(u_f, v_f, edge_index):
    src = edge_index[0].astype(jnp.int32)
    dst = edge_index[1].astype(jnp.int32)
    node_f = jnp.concatenate([u_f, v_f], axis=0)
    zeros_2d = jnp.zeros((N_NODES, D), jnp.float32)

    sdeg80 = _bincount_tc(src.reshape(E_TOTAL, 1))
    sdeg = sdeg80.reshape(DEG_HI * 128)[:N_NODES].reshape(N_NODES, 1)

    table = _scale(node_f, sdeg)
    dst3 = dst.reshape(NW, NSTEP, CHUNK)
    parts = _message_kernel(table, src, dst3, zeros_2d)
    parts = parts.reshape(NC, N_NODES, D)

    ddeg80 = _bincount_tc(dst.reshape(E_TOTAL, 1))
    ddeg = ddeg80.reshape(DEG_HI * 128)[:N_NODES].reshape(N_NODES, 1)
    return _finish(parts, ddeg)


# speedup vs baseline: 1.0745x; 1.0745x over previous
import functools

import jax
import jax.numpy as jnp
from jax import lax
from jax.experimental import pallas as pl
from jax.experimental.pallas import tpu as pltpu
from jax.experimental.pallas import tpu_sc as plsc

N_NODES = 10000
D = 128
E_TOTAL = 320000
NC = 2
NS = 16
NW = NC * NS
EPW = E_TOTAL // NW
CHUNK = 80
NSTEP = EPW // CHUNK
ROW_STRIDE = 624
COPY_ROWS = 640

_mesh = plsc.VectorSubcoreMesh(core_axis_name="c", subcore_axis_name="s")


def _make_pass_kernel(width):
    @functools.partial(
        pl.kernel,
        out_type=jax.ShapeDtypeStruct((NC * N_NODES, width), jnp.float32),
        mesh=_mesh,
        scratch_types=[
            pltpu.VMEM((EPW,), jnp.int32),
            pltpu.VMEM((NSTEP, CHUNK), jnp.int32),
            pltpu.VMEM((CHUNK, width), jnp.float32),
            pltpu.VMEM((CHUNK, width), jnp.float32),
            pltpu.VMEM_SHARED((N_NODES, width), jnp.float32),
            pltpu.SemaphoreType.DMA,
            pltpu.SemaphoreType.DMA,
        ],
    )
    def pass_kernel(table_hbm, src_hbm, dst_hbm, zrows_hbm, out_hbm,
                    sidx, didx, rows0, rows1, accum, sem0, sem1):
        c = lax.axis_index("c")
        s = lax.axis_index("s")
        wid = s * NC + c
        r0 = pl.multiple_of(s * ROW_STRIDE, 8)

        pltpu.sync_copy(zrows_hbm.at[pl.ds(r0, COPY_ROWS)],
                        accum.at[pl.ds(r0, COPY_ROWS)])
        base = pl.multiple_of(wid * EPW, 8)
        pltpu.sync_copy(src_hbm.at[pl.ds(base, EPW)], sidx)
        pltpu.sync_copy(dst_hbm.at[wid], didx)
        plsc.subcore_barrier()

        def start_gather(j, rows, sem):
            pltpu.async_copy(
                table_hbm.at[sidx.at[pl.ds(j * CHUNK, CHUNK)]], rows, sem)

        def wait_gather(rows, sem):
            pltpu.make_async_copy(table_hbm.at[pl.ds(0, CHUNK)], rows,
                                  sem).wait()

        start_gather(0, rows0, sem0)

        def pair(k, carry):
            j0 = k * 2
            wait_gather(rows0, sem0)
            start_gather(j0 + 1, rows1, sem1)
            pltpu.sync_copy(rows0, accum.at[didx.at[j0]], add=True)
            wait_gather(rows1, sem1)
            start_gather(j0 + 2, rows0, sem0)
            pltpu.sync_copy(rows1, accum.at[didx.at[j0 + 1]], add=True)
            return carry

        lax.fori_loop(0, NSTEP // 2, pair, 0)
        wait_gather(rows0, sem0)
        pltpu.sync_copy(rows0, accum.at[didx.at[NSTEP - 1]], add=True)

        plsc.subcore_barrier()
        pltpu.sync_copy(
            accum.at[pl.ds(r0, COPY_ROWS)],
            out_hbm.at[pl.ds(c * N_NODES + r0, COPY_ROWS)])

    return pass_kernel


_message_kernel = _make_pass_kernel(D)

DEG_HI = 80
CH_E = 16000
NCH = E_TOTAL // CH_E


def _bincount_body(idx_ref, deg_ref):
    i = pl.program_id(0)

    @pl.when(i == 0)
    def _():
        deg_ref[...] = jnp.zeros_like(deg_ref)

    idx_col = idx_ref[...]
    hi = idx_col // 128
    lo = idx_col - hi * 128
    a = (lo == lax.broadcasted_iota(jnp.int32, (CH_E, 128), 1))
    bt = (hi == lax.broadcasted_iota(jnp.int32, (CH_E, DEG_HI), 1))
    deg_ref[...] += lax.dot_general(
        bt.astype(jnp.bfloat16), a.astype(jnp.bfloat16),
        (((0,), (0,)), ((), ())), preferred_element_type=jnp.float32)


_bincount_tc = pl.pallas_call(
    _bincount_body,
    grid=(NCH,),
    in_specs=[pl.BlockSpec((CH_E, 1), lambda i: (i, 0))],
    out_specs=pl.BlockSpec((DEG_HI, 128), lambda i: (0, 0)),
    out_shape=jax.ShapeDtypeStruct((DEG_HI, 128), jnp.float32),
)


def _scale_body(node_ref, deg_ref, out_ref):
    deg = jnp.maximum(deg_ref[...], 1.0)
    out_ref[...] = node_ref[...] * lax.rsqrt(deg)


_scale = pl.pallas_call(
    _scale_body,
    out_shape=jax.ShapeDtypeStruct((N_NODES, D), jnp.float32),
)


def _finish_body(part_ref, deg_ref, out_ref):
    total = part_ref[0] + part_ref[1]
    deg = jnp.maximum(deg_ref[...], 1.0)
    out_ref[...] = total * lax.rsqrt(deg)


_finish = pl.pallas_call(
    _finish_body,
    out_shape=jax.ShapeDtypeStruct((N_NODES, D), jnp.float32),
)


def kernel(u_f, v_f, edge_index):
    src = edge_index[0].astype(jnp.int32)
    dst = edge_index[1].astype(jnp.int32)
    node_f = jnp.concatenate([u_f, v_f], axis=0)
    zeros_2d = jnp.zeros((N_NODES, D), jnp.float32)

    sdeg80 = _bincount_tc(src.reshape(E_TOTAL, 1))
    sdeg = sdeg80.reshape(DEG_HI * 128)[:N_NODES].reshape(N_NODES, 1)

    table = _scale(node_f, sdeg)
    dst3 = dst.reshape(NW, NSTEP, CHUNK)
    parts = _message_kernel(table, src, dst3, zeros_2d)
    parts = parts.reshape(NC, N_NODES, D)

    ddeg80 = _bincount_tc(dst.reshape(E_TOTAL, 1))
    ddeg = ddeg80.reshape(DEG_HI * 128)[:N_NODES].reshape(N_NODES, 1)
    return _finish(parts, ddeg)
